# baseline (speedup 1.0000x reference)
import functools

import jax
import jax.numpy as jnp
from jax import lax
from jax.experimental import pallas as pl
from jax.experimental.pallas import tpu as pltpu
from jax.experimental.pallas import tpu_sc as plsc

B = 16384
MAX_LEN = 50
VOCAB = 1000000
EMBED = 32

N = B * MAX_LEN
NW = 32
NPW = N // NW
BCH = 32
CH = BCH * MAX_LEN
NCHUNK = NPW // CH
BPW = B // NW
ZROW = VOCAB

_mesh = plsc.VectorSubcoreMesh(core_axis_name="c", subcore_axis_name="s")


@functools.partial(
    pl.kernel,
    mesh=_mesh,
    compiler_params=pltpu.CompilerParams(use_tc_tiling_on_sc=False),
    out_type=jax.ShapeDtypeStruct((N, EMBED), jnp.float32),
    scratch_types=[
        pltpu.VMEM((CH,), jnp.int32),
        pltpu.VMEM((CH, EMBED), jnp.float32),
        pltpu.VMEM((BPW + 16,), jnp.int32),
        pltpu.SemaphoreType.DMA,
    ],
)
def _embed_sc(idx_hbm, seq_hbm, table_hbm, out_hbm, idx_v, rows_v, seq_v, sem):
    wid = lax.axis_index("s") * 2 + lax.axis_index("c")
    base = wid * NPW
    pltpu.sync_copy(seq_hbm.at[pl.ds(wid * BPW, BPW)], seq_v.at[pl.ds(0, BPW)])

    def chunk_body(g, _):
        off = base + g * CH
        pltpu.sync_copy(idx_hbm.at[pl.ds(off, CH)], idx_v)

        def fix_body(brel, _):
            sl = seq_v[pl.ds(g * BCH + brel, 16)][0]
            rbase = brel * MAX_LEN
            for o in (0, 16, 32, 34):
                t = o + lax.iota(jnp.int32, 16)
                iv = idx_v[pl.ds(rbase + o, 16)]
                idx_v[pl.ds(rbase + o, 16)] = jnp.where(t < sl, iv, ZROW)
            return 0

        lax.fori_loop(0, BCH, fix_body, 0)

        pltpu.async_copy(table_hbm.at[idx_v], rows_v, sem).wait()
        pltpu.sync_copy(rows_v, out_hbm.at[pl.ds(off, CH)])
        return 0

    lax.fori_loop(0, NCHUNK, chunk_body, 0)


def kernel(indices, seq_lens, table):
    idx = indices.astype(jnp.int32).reshape(-1)
    seq = seq_lens.astype(jnp.int32)
    table_ext = jnp.concatenate(
        [table, jnp.zeros((8, EMBED), table.dtype)], axis=0
    )
    out = _embed_sc(idx, seq, table_ext)
    return out.reshape(B, MAX_LEN * EMBED)


# speedup vs baseline: 4.5779x; 4.5779x over previous
import functools

import jax
import jax.numpy as jnp
from jax import lax
from jax.experimental import pallas as pl
from jax.experimental.pallas import tpu as pltpu
from jax.experimental.pallas import tpu_sc as plsc

B = 16384
MAX_LEN = 50
VOCAB = 1000000
EMBED = 32

N = B * MAX_LEN
NW = 32
NPW = N // NW
BCH = 32
CH = BCH * MAX_LEN
NCHUNK = NPW // CH
BPW = B // NW
NPAD = 2048
ZROW = VOCAB

_mesh = plsc.VectorSubcoreMesh(core_axis_name="c", subcore_axis_name="s")


@functools.partial(
    pl.kernel,
    mesh=_mesh,
    compiler_params=pltpu.CompilerParams(use_tc_tiling_on_sc=False),
    out_type=jax.ShapeDtypeStruct((N, EMBED), jnp.float32),
    scratch_types=[
        pltpu.VMEM((CH,), jnp.int32),
        pltpu.VMEM((CH, EMBED), jnp.float32),
        pltpu.VMEM((BPW + 16,), jnp.int32),
        pltpu.SemaphoreType.DMA,
    ],
)
def _embed_sc(idx_hbm, seq_hbm, table_hbm, out_hbm, idx_v, rows_v, seq_v, sem):
    wid = lax.axis_index("s") * 2 + lax.axis_index("c")
    base = wid * NPW
    pltpu.sync_copy(seq_hbm.at[pl.ds(wid * BPW, BPW)], seq_v.at[pl.ds(0, BPW)])

    def chunk_body(g, _):
        off = base + g * CH
        pltpu.sync_copy(idx_hbm.at[pl.ds(off, CH)], idx_v)

        def fix_body(brel, _):
            sl = seq_v[pl.ds(g * BCH + brel, 16)][0]
            rbase = brel * MAX_LEN
            pad = wid * 64 + brel + lax.iota(jnp.int32, 16)
            pad = ZROW + lax.rem(pad * 37, NPAD)
            for o in (0, 16, 32, 34):
                t = o + lax.iota(jnp.int32, 16)
                iv = idx_v[pl.ds(rbase + o, 16)]
                idx_v[pl.ds(rbase + o, 16)] = jnp.where(t < sl, iv, pad)
            return 0

        lax.fori_loop(0, BCH, fix_body, 0)

        pltpu.async_copy(table_hbm.at[idx_v], rows_v, sem).wait()
        pltpu.sync_copy(rows_v, out_hbm.at[pl.ds(off, CH)])
        return 0

    lax.fori_loop(0, NCHUNK, chunk_body, 0)


def kernel(indices, seq_lens, table):
    idx = indices.astype(jnp.int32).reshape(-1)
    seq = seq_lens.astype(jnp.int32)
    table_ext = jnp.concatenate(
        [table, jnp.zeros((NPAD, EMBED), table.dtype)], axis=0
    )
    out = _embed_sc(idx, seq, table_ext)
    return out.reshape(B, MAX_LEN * EMBED)


# speedup vs baseline: 6.1330x; 1.3397x over previous
import functools

import jax
import jax.numpy as jnp
from jax import lax
from jax.experimental import pallas as pl
from jax.experimental.pallas import tpu as pltpu
from jax.experimental.pallas import tpu_sc as plsc

B = 16384
MAX_LEN = 50
VOCAB = 1000000
EMBED = 32

N = B * MAX_LEN
NW = 32
NPW = N // NW
BCH = 32
CH = BCH * MAX_LEN
NCHUNK = NPW // CH
BPW = B // NW

_mesh = plsc.VectorSubcoreMesh(core_axis_name="c", subcore_axis_name="s")


@functools.partial(
    pl.kernel,
    mesh=_mesh,
    compiler_params=pltpu.CompilerParams(use_tc_tiling_on_sc=False),
    out_type=jax.ShapeDtypeStruct((N, EMBED), jnp.float32),
    scratch_types=[
        pltpu.VMEM((CH,), jnp.int32),
        pltpu.VMEM((CH, EMBED), jnp.float32),
        pltpu.VMEM((BPW + 16,), jnp.int32),
        pltpu.SemaphoreType.DMA,
    ],
)
def _embed_sc(idx_hbm, seq_hbm, table_hbm, out_hbm, idx_v, rows_v, seq_v, sem):
    wid = lax.axis_index("s") * 2 + lax.axis_index("c")
    base = wid * NPW
    pltpu.sync_copy(seq_hbm.at[pl.ds(wid * BPW, BPW)], seq_v.at[pl.ds(0, BPW)])
    zvec = jnp.zeros((16,), jnp.float32)

    def chunk_body(g, _):
        off = base + g * CH
        pltpu.sync_copy(idx_hbm.at[pl.ds(off, CH)], idx_v)
        pltpu.async_copy(table_hbm.at[idx_v], rows_v, sem).wait()

        def zero_b(brel, _):
            sl = seq_v[pl.ds(g * BCH + brel, 16)][0]

            def zero_row(r, _):
                rows_v[brel * MAX_LEN + r, pl.ds(0, 16)] = zvec
                rows_v[brel * MAX_LEN + r, pl.ds(16, 16)] = zvec
                return 0

            lax.fori_loop(sl, MAX_LEN, zero_row, 0)
            return 0

        lax.fori_loop(0, BCH, zero_b, 0)

        pltpu.sync_copy(rows_v, out_hbm.at[pl.ds(off, CH)])
        return 0

    lax.fori_loop(0, NCHUNK, chunk_body, 0)


def kernel(indices, seq_lens, table):
    idx = indices.astype(jnp.int32).reshape(-1)
    seq = seq_lens.astype(jnp.int32)
    out = _embed_sc(idx, seq, table)
    return out.reshape(B, MAX_LEN * EMBED)
